# baseline (device time: 39846 ns/iter reference)
import jax
import jax.numpy as jnp
from jax import lax
from jax.experimental import pallas as pl
from jax.experimental.pallas import tpu as pltpu

CHUNK_ROWS = (256, 256, 192, 128, 96, 48, 24, 16, 8)
C = len(CHUNK_ROWS)


def kernel(x):
    m, n = x.shape
    m2 = m // 2
    assert sum(CHUNK_ROWS) == m2
    offs = [sum(CHUNK_ROWS[:c]) for c in range(C)]

    def body(x_ref, out_ref, p1_send, p1_recv, p2_send, p2_recv, loc_sem):
        my_x = lax.axis_index("x")
        my_y = lax.axis_index("y")
        nbr_y = 1 - my_y
        nbr_x = 1 - my_x
        half = my_x * m2

        barrier_sem = pltpu.get_barrier_semaphore()
        pl.semaphore_signal(
            barrier_sem, inc=1,
            device_id=(my_x, nbr_y), device_id_type=pl.DeviceIdType.MESH,
        )
        pl.semaphore_signal(
            barrier_sem, inc=1,
            device_id=(nbr_x, my_y), device_id_type=pl.DeviceIdType.MESH,
        )
        pl.semaphore_wait(barrier_sem, 2)

        p1 = []
        for c in range(C):
            o, r = offs[c], CHUNK_ROWS[c]
            rdma = pltpu.make_async_remote_copy(
                src_ref=x_ref.at[pl.ds(half + o, r), :],
                dst_ref=out_ref.at[pl.ds(my_y * m + half + o, r), :],
                send_sem=p1_send.at[c],
                recv_sem=p1_recv.at[c],
                device_id=(my_x, nbr_y),
                device_id_type=pl.DeviceIdType.MESH,
            )
            rdma.start()
            p1.append(rdma)

        loc = pltpu.make_async_copy(
            x_ref, out_ref.at[pl.ds(my_y * m, m), :], loc_sem
        )
        loc.start()

        p2 = []
        for c in range(C):
            o, r = offs[c], CHUNK_ROWS[c]
            inb = out_ref.at[pl.ds(nbr_y * m + half + o, r), :]
            recv = pltpu.make_async_remote_copy(
                src_ref=x_ref.at[pl.ds(o, r), :],
                dst_ref=inb,
                send_sem=p1_send.at[c],
                recv_sem=p1_recv.at[c],
                device_id=(my_x, nbr_y),
                device_id_type=pl.DeviceIdType.MESH,
            )
            recv.wait_recv()
            fwd = pltpu.make_async_remote_copy(
                src_ref=inb,
                dst_ref=inb,
                send_sem=p2_send.at[c],
                recv_sem=p2_recv.at[c],
                device_id=(nbr_x, my_y),
                device_id_type=pl.DeviceIdType.MESH,
            )
            fwd.start()
            p2.append(fwd)

        for c in range(C):
            o, r = offs[c], CHUNK_ROWS[c]
            inb2 = out_ref.at[pl.ds(nbr_y * m + nbr_x * m2 + o, r), :]
            recv2 = pltpu.make_async_remote_copy(
                src_ref=x_ref.at[pl.ds(o, r), :],
                dst_ref=inb2,
                send_sem=p2_send.at[c],
                recv_sem=p2_recv.at[c],
                device_id=(nbr_x, my_y),
                device_id_type=pl.DeviceIdType.MESH,
            )
            recv2.wait_recv()
        for c in range(C):
            p1[c].wait_send()
            p2[c].wait_send()
        loc.wait()

    return pl.pallas_call(
        body,
        out_shape=jax.ShapeDtypeStruct((2 * m, n), x.dtype),
        in_specs=[pl.BlockSpec(memory_space=pltpu.VMEM)],
        out_specs=pl.BlockSpec(memory_space=pltpu.VMEM),
        scratch_shapes=[
            pltpu.SemaphoreType.DMA((C,)),
            pltpu.SemaphoreType.DMA((C,)),
            pltpu.SemaphoreType.DMA((C,)),
            pltpu.SemaphoreType.DMA((C,)),
            pltpu.SemaphoreType.DMA,
        ],
        compiler_params=pltpu.CompilerParams(collective_id=0),
    )(x)


# device time: 35783 ns/iter; 1.1135x vs baseline; 1.1135x over previous
import jax
import jax.numpy as jnp
from jax import lax
from jax.experimental import pallas as pl
from jax.experimental.pallas import tpu as pltpu

C = 32
CHUNK_ROWS = (1024 // C,) * C


def kernel(x):
    m, n = x.shape
    m2 = m // 2
    assert sum(CHUNK_ROWS) == m2
    offs = [sum(CHUNK_ROWS[:c]) for c in range(C)]

    def body(x_ref, out_ref, p1_send, p1_recv, p2_send, p2_recv, loc_sem):
        my_x = lax.axis_index("x")
        my_y = lax.axis_index("y")
        nbr_y = 1 - my_y
        nbr_x = 1 - my_x
        half = my_x * m2

        barrier_sem = pltpu.get_barrier_semaphore()
        pl.semaphore_signal(
            barrier_sem, inc=1,
            device_id=(my_x, nbr_y), device_id_type=pl.DeviceIdType.MESH,
        )
        pl.semaphore_signal(
            barrier_sem, inc=1,
            device_id=(nbr_x, my_y), device_id_type=pl.DeviceIdType.MESH,
        )
        pl.semaphore_wait(barrier_sem, 2)

        p1 = []
        for c in range(C):
            o, r = offs[c], CHUNK_ROWS[c]
            rdma = pltpu.make_async_remote_copy(
                src_ref=x_ref.at[pl.ds(half + o, r), :],
                dst_ref=out_ref.at[pl.ds(my_y * m + half + o, r), :],
                send_sem=p1_send.at[c],
                recv_sem=p1_recv.at[c],
                device_id=(my_x, nbr_y),
                device_id_type=pl.DeviceIdType.MESH,
            )
            rdma.start()
            p1.append(rdma)

        loc = pltpu.make_async_copy(
            x_ref, out_ref.at[pl.ds(my_y * m, m), :], loc_sem
        )
        loc.start()

        p2 = []
        for c in range(C):
            o, r = offs[c], CHUNK_ROWS[c]
            inb = out_ref.at[pl.ds(nbr_y * m + half + o, r), :]
            recv = pltpu.make_async_remote_copy(
                src_ref=x_ref.at[pl.ds(o, r), :],
                dst_ref=inb,
                send_sem=p1_send.at[c],
                recv_sem=p1_recv.at[c],
                device_id=(my_x, nbr_y),
                device_id_type=pl.DeviceIdType.MESH,
            )
            recv.wait_recv()
            fwd = pltpu.make_async_remote_copy(
                src_ref=inb,
                dst_ref=inb,
                send_sem=p2_send.at[c],
                recv_sem=p2_recv.at[c],
                device_id=(nbr_x, my_y),
                device_id_type=pl.DeviceIdType.MESH,
            )
            fwd.start()
            p2.append(fwd)

        for c in range(C):
            o, r = offs[c], CHUNK_ROWS[c]
            inb2 = out_ref.at[pl.ds(nbr_y * m + nbr_x * m2 + o, r), :]
            recv2 = pltpu.make_async_remote_copy(
                src_ref=x_ref.at[pl.ds(o, r), :],
                dst_ref=inb2,
                send_sem=p2_send.at[c],
                recv_sem=p2_recv.at[c],
                device_id=(nbr_x, my_y),
                device_id_type=pl.DeviceIdType.MESH,
            )
            recv2.wait_recv()
        for c in range(C):
            p1[c].wait_send()
            p2[c].wait_send()
        loc.wait()

    return pl.pallas_call(
        body,
        out_shape=jax.ShapeDtypeStruct((2 * m, n), x.dtype),
        in_specs=[pl.BlockSpec(memory_space=pltpu.VMEM)],
        out_specs=pl.BlockSpec(memory_space=pltpu.VMEM),
        scratch_shapes=[
            pltpu.SemaphoreType.DMA((C,)),
            pltpu.SemaphoreType.DMA((C,)),
            pltpu.SemaphoreType.DMA((C,)),
            pltpu.SemaphoreType.DMA((C,)),
            pltpu.SemaphoreType.DMA,
        ],
        compiler_params=pltpu.CompilerParams(collective_id=0),
    )(x)
